# baseline (speedup 1.0000x reference)
import functools

import jax
import jax.numpy as jnp
from jax import lax
from jax.experimental import pallas as pl
from jax.experimental.pallas import tpu as pltpu
from jax.experimental.pallas import tpu_sc as plsc

EMBED_DIM = 64
BATCH = 16384
NUM_CORES = 2
NUM_SUBCORES = 16
NUM_WORKERS = NUM_CORES * NUM_SUBCORES
B_PER_W = BATCH // NUM_WORKERS

_MESH = plsc.VectorSubcoreMesh(
    core_axis_name="c", subcore_axis_name="s",
    num_cores=NUM_CORES, num_subcores=NUM_SUBCORES,
)


@functools.partial(
    pl.kernel,
    out_type=jax.ShapeDtypeStruct((BATCH, EMBED_DIM), jnp.float32),
    mesh=_MESH,
    scratch_types=[
        pltpu.VMEM((B_PER_W,), jnp.int32),
        pltpu.VMEM((B_PER_W, EMBED_DIM), jnp.float32),
        pltpu.SemaphoreType.DMA,
    ],
)
def _sc_gather(table_hbm, idx_hbm, out_hbm, idx_v, rows_v, sem):
    wid = lax.axis_index("s") * NUM_CORES + lax.axis_index("c")
    base = wid * B_PER_W
    pltpu.sync_copy(idx_hbm.at[pl.ds(base, B_PER_W)], idx_v)

    def fire(g, _):
        vec = idx_v[pl.ds(g * 16, 16)]
        for j in range(16):
            pltpu.make_async_copy(
                table_hbm.at[vec[j]], rows_v.at[g * 16 + j], sem
            ).start()
        return _

    def drain(g, _):
        vec = idx_v[pl.ds(g * 16, 16)]
        for j in range(16):
            pltpu.make_async_copy(
                table_hbm.at[vec[j]], rows_v.at[g * 16 + j], sem
            ).wait()
        return _

    lax.fori_loop(0, B_PER_W // 16, fire, None)
    lax.fori_loop(0, B_PER_W // 16, drain, None)
    pltpu.sync_copy(rows_v, out_hbm.at[pl.ds(base, B_PER_W)])


def _copy2_body(x_ref, o1_ref, o2_ref):
    x = x_ref[...]
    o1_ref[...] = x
    o2_ref[...] = x


_copy2 = pl.pallas_call(
    _copy2_body,
    grid=(8,),
    in_specs=[pl.BlockSpec((BATCH // 8, EMBED_DIM), lambda i: (i, 0))],
    out_specs=[pl.BlockSpec((BATCH // 8, EMBED_DIM), lambda i: (i, 0))] * 2,
    out_shape=[jax.ShapeDtypeStruct((BATCH, EMBED_DIM), jnp.float32)] * 2,
)


def kernel(node_embedding, node_label, current_context, embeddings_weight):
    idx = node_label.astype(jnp.int32)
    node_label_ = _sc_gather(embeddings_weight, idx)
    out1, out2 = _copy2(node_embedding)
    return (out1, out2, node_label_)


# speedup vs baseline: 1.0674x; 1.0674x over previous
import functools

import jax
import jax.numpy as jnp
from jax import lax
from jax.experimental import pallas as pl
from jax.experimental.pallas import tpu as pltpu
from jax.experimental.pallas import tpu_sc as plsc

EMBED_DIM = 64
BATCH = 16384
OP_ROWS = 1_000_000
NUM_CORES = 2
NUM_SUBCORES = 16
NUM_WORKERS = NUM_CORES * NUM_SUBCORES
B_PER_W = BATCH // NUM_WORKERS

_MESH = plsc.VectorSubcoreMesh(
    core_axis_name="c", subcore_axis_name="s",
    num_cores=NUM_CORES, num_subcores=NUM_SUBCORES,
)


@functools.partial(
    pl.kernel,
    out_type=jax.ShapeDtypeStruct((BATCH, EMBED_DIM), jnp.float32),
    mesh=_MESH,
    scratch_types=[
        pltpu.VMEM((B_PER_W,), jnp.int32),
        pltpu.VMEM((B_PER_W, EMBED_DIM), jnp.float32),
        pltpu.SemaphoreType.DMA,
    ],
)
def _sc_gather(table_hbm, idx_hbm, out_hbm, idx_v, rows_v, sem):
    wid = lax.axis_index("s") * NUM_CORES + lax.axis_index("c")
    base = wid * B_PER_W
    pltpu.sync_copy(idx_hbm.at[pl.ds(base, B_PER_W)], idx_v)

    def fire(g, _):
        vec = idx_v[pl.ds(g * 16, 16)]
        for j in range(16):
            pltpu.make_async_copy(
                table_hbm.at[vec[j]], rows_v.at[g * 16 + j], sem
            ).start()
        return _

    def drain(g, _):
        vec = idx_v[pl.ds(g * 16, 16)]
        for j in range(16):
            pltpu.make_async_copy(
                table_hbm.at[vec[j]], rows_v.at[g * 16 + j], sem
            ).wait()
        return _

    lax.fori_loop(0, B_PER_W // 16, fire, None)
    lax.fori_loop(0, B_PER_W // 16, drain, None)
    pltpu.sync_copy(rows_v, out_hbm.at[pl.ds(base, B_PER_W)])


_TR_LANES = 4096


def _transpose_body(x_ref, o_ref):
    o_ref[...] = x_ref[...].T


_transpose = pl.pallas_call(
    _transpose_body,
    grid=(pl.cdiv(OP_ROWS, _TR_LANES),),
    in_specs=[pl.BlockSpec((EMBED_DIM, _TR_LANES), lambda i: (0, i))],
    out_specs=pl.BlockSpec((_TR_LANES, EMBED_DIM), lambda i: (i, 0)),
    out_shape=jax.ShapeDtypeStruct((OP_ROWS, EMBED_DIM), jnp.float32),
)


def _copy2_body(x_ref, o1_ref, o2_ref):
    x = x_ref[...]
    o1_ref[...] = x
    o2_ref[...] = x


_copy2 = pl.pallas_call(
    _copy2_body,
    grid=(8,),
    in_specs=[pl.BlockSpec((EMBED_DIM, BATCH // 8), lambda i: (0, i))],
    out_specs=[pl.BlockSpec((EMBED_DIM, BATCH // 8), lambda i: (0, i))] * 2,
    out_shape=[jax.ShapeDtypeStruct((EMBED_DIM, BATCH), jnp.float32)] * 2,
)


def kernel(node_embedding, node_label, current_context, embeddings_weight):
    idx = node_label.astype(jnp.int32)
    table_rm = _transpose(embeddings_weight.T)
    out = _sc_gather(table_rm, idx)
    o1T, o2T = _copy2(node_embedding.T)
    return (o1T.T, o2T.T, out)


# speedup vs baseline: 1.3055x; 1.2230x over previous
import functools

import jax
import jax.numpy as jnp
from jax import lax
from jax.experimental import pallas as pl
from jax.experimental.pallas import tpu as pltpu
from jax.experimental.pallas import tpu_sc as plsc

EMBED_DIM = 64
BATCH = 16384
OP_ROWS = 1_000_000
NUM_CORES = 2
NUM_SUBCORES = 16
NUM_WORKERS = NUM_CORES * NUM_SUBCORES
B_PER_W = BATCH // NUM_WORKERS
CHUNK = 128
NUM_CHUNKS = B_PER_W // CHUNK

_TR_L = 4096
_GRID = 123
_S_HALF = _TR_L * _GRID

_MESH = plsc.VectorSubcoreMesh(
    core_axis_name="c", subcore_axis_name="s",
    num_cores=NUM_CORES, num_subcores=NUM_SUBCORES,
)


def _stage_body(xa_ref, xb_ref, o_ref):
    o_ref[...] = jnp.concatenate([xa_ref[...].T, xb_ref[...].T], axis=1)


_stage = pl.pallas_call(
    _stage_body,
    grid=(_GRID,),
    in_specs=[
        pl.BlockSpec((EMBED_DIM, _TR_L), lambda i: (0, i)),
        pl.BlockSpec((EMBED_DIM, _TR_L),
                     lambda i: (0, jnp.minimum(i + _GRID, 2 * _GRID - 2))),
    ],
    out_specs=pl.BlockSpec((_TR_L, 2 * EMBED_DIM), lambda i: (i, 0)),
    out_shape=jax.ShapeDtypeStruct((_S_HALF, 2 * EMBED_DIM), jnp.float32),
)


@functools.partial(
    pl.kernel,
    out_type=jax.ShapeDtypeStruct((BATCH * EMBED_DIM,), jnp.float32),
    mesh=_MESH,
    scratch_types=[
        pltpu.VMEM((B_PER_W,), jnp.int32),
        pltpu.VMEM((B_PER_W * 2 * EMBED_DIM,), jnp.float32),
        pltpu.VMEM((B_PER_W * EMBED_DIM,), jnp.float32),
        pltpu.SemaphoreType.DMA,
    ],
)
def _sc_gather(staged_hbm, idx_hbm, out_hbm,
               idx_v, pairs_flat, rows_flat, sem):
    wid = lax.axis_index("s") * NUM_CORES + lax.axis_index("c")
    base = wid * B_PER_W
    pltpu.sync_copy(idx_hbm.at[pl.ds(base, B_PER_W)], idx_v)

    def fire(g, carry):
        vec = idx_v[pl.ds(g * 16, 16)]
        rvec = jnp.where(vec >= _S_HALF, vec - _S_HALF, vec)
        for j in range(16):
            k = g * 16 + j
            pltpu.make_async_copy(
                staged_hbm.at[rvec[j]],
                pairs_flat.at[pl.ds(k * 2 * EMBED_DIM, 2 * EMBED_DIM)],
                sem,
            ).start()
        return carry

    def drain(g, carry):
        vec = idx_v[pl.ds(g * 16, 16)]
        rvec = jnp.where(vec >= _S_HALF, vec - _S_HALF, vec)
        for j in range(16):
            k = g * 16 + j
            pltpu.make_async_copy(
                staged_hbm.at[rvec[j]],
                pairs_flat.at[pl.ds(k * 2 * EMBED_DIM, 2 * EMBED_DIM)],
                sem,
            ).wait()
        return carry

    lax.fori_loop(0, B_PER_W // 16, fire, None)
    lax.fori_loop(0, B_PER_W // 16, drain, None)

    def extract(g, carry):
        vec = idx_v[pl.ds(g * 16, 16)]
        avec = jnp.where(vec >= _S_HALF, EMBED_DIM, 0)
        for j in range(16):
            k = g * 16 + j
            a = avec[j]
            for q in range(EMBED_DIM // 16):
                vals = pairs_flat[pl.ds(k * 2 * EMBED_DIM + a + 16 * q, 16)]
                rows_flat[pl.ds(k * EMBED_DIM + 16 * q, 16)] = vals
        return carry

    lax.fori_loop(0, B_PER_W // 16, extract, None)
    pltpu.sync_copy(rows_flat, out_hbm.at[pl.ds(base * EMBED_DIM,
                                                B_PER_W * EMBED_DIM)])


def _copy2_body(x_ref, o1_ref, o2_ref):
    x = x_ref[...]
    o1_ref[...] = x
    o2_ref[...] = x


_copy2 = pl.pallas_call(
    _copy2_body,
    grid=(8,),
    in_specs=[pl.BlockSpec((EMBED_DIM, BATCH // 8), lambda i: (0, i))],
    out_specs=[pl.BlockSpec((EMBED_DIM, BATCH // 8), lambda i: (0, i))] * 2,
    out_shape=[jax.ShapeDtypeStruct((EMBED_DIM, BATCH), jnp.float32)] * 2,
)


def kernel(node_embedding, node_label, current_context, embeddings_weight):
    idx = node_label.astype(jnp.int32)
    table_t = embeddings_weight.T
    staged = _stage(table_t, table_t)
    out_flat = _sc_gather(staged, idx)
    o1T, o2T = _copy2(node_embedding.T)
    return (o1T.T, o2T.T, out_flat.reshape(BATCH, EMBED_DIM))


# speedup vs baseline: 1.4643x; 1.1216x over previous
import functools

import jax
import jax.numpy as jnp
from jax import lax
from jax.experimental import pallas as pl
from jax.experimental.pallas import tpu as pltpu
from jax.experimental.pallas import tpu_sc as plsc

EMBED_DIM = 64
BATCH = 16384
OP_ROWS = 1_000_000
NUM_CORES = 2
NUM_SUBCORES = 16
NUM_WORKERS = NUM_CORES * NUM_SUBCORES
B_PER_W = BATCH // NUM_WORKERS
CHUNK = 128
NUM_CHUNKS = B_PER_W // CHUNK

_TR_L = 8192
_GRID = 62
_S_HALF = _TR_L * _GRID

_MESH = plsc.VectorSubcoreMesh(
    core_axis_name="c", subcore_axis_name="s",
    num_cores=NUM_CORES, num_subcores=NUM_SUBCORES,
)


def _stage_body(xa_ref, xb_ref, o_ref):
    o_ref[...] = jnp.concatenate([xa_ref[...].T, xb_ref[...].T], axis=1)


_stage = pl.pallas_call(
    _stage_body,
    grid=(_GRID,),
    in_specs=[
        pl.BlockSpec((EMBED_DIM, _TR_L), lambda i: (0, i)),
        pl.BlockSpec((EMBED_DIM, _TR_L),
                     lambda i: (0, jnp.minimum(i + _GRID, 2 * _GRID - 2))),
    ],
    out_specs=pl.BlockSpec((_TR_L, 2 * EMBED_DIM), lambda i: (i, 0)),
    out_shape=jax.ShapeDtypeStruct((_S_HALF, 2 * EMBED_DIM), jnp.float32),
)


@functools.partial(
    pl.kernel,
    out_type=jax.ShapeDtypeStruct((BATCH * EMBED_DIM,), jnp.float32),
    mesh=_MESH,
    scratch_types=[
        pltpu.VMEM((B_PER_W,), jnp.int32),
        pltpu.VMEM((B_PER_W * 2 * EMBED_DIM,), jnp.float32),
        pltpu.VMEM((B_PER_W * EMBED_DIM,), jnp.float32),
        pltpu.SemaphoreType.DMA,
    ],
)
def _sc_gather(staged_hbm, idx_hbm, out_hbm,
               idx_v, pairs_flat, rows_flat, sem):
    wid = lax.axis_index("s") * NUM_CORES + lax.axis_index("c")
    base = wid * B_PER_W
    pltpu.sync_copy(idx_hbm.at[pl.ds(base, B_PER_W)], idx_v)

    def fire(g, carry):
        vec = idx_v[pl.ds(g * 16, 16)]
        rvec = jnp.where(vec >= _S_HALF, vec - _S_HALF, vec)
        for j in range(16):
            k = g * 16 + j
            pltpu.make_async_copy(
                staged_hbm.at[rvec[j]],
                pairs_flat.at[pl.ds(k * 2 * EMBED_DIM, 2 * EMBED_DIM)],
                sem,
            ).start()
        return carry

    def drain(g, carry):
        vec = idx_v[pl.ds(g * 16, 16)]
        rvec = jnp.where(vec >= _S_HALF, vec - _S_HALF, vec)
        for j in range(16):
            k = g * 16 + j
            pltpu.make_async_copy(
                staged_hbm.at[rvec[j]],
                pairs_flat.at[pl.ds(k * 2 * EMBED_DIM, 2 * EMBED_DIM)],
                sem,
            ).wait()
        return carry

    lax.fori_loop(0, B_PER_W // 16, fire, None)
    lax.fori_loop(0, B_PER_W // 16, drain, None)

    def extract(g, carry):
        vec = idx_v[pl.ds(g * 16, 16)]
        avec = jnp.where(vec >= _S_HALF, EMBED_DIM, 0)
        for j in range(16):
            k = g * 16 + j
            a = avec[j]
            for q in range(EMBED_DIM // 16):
                vals = pairs_flat[pl.ds(k * 2 * EMBED_DIM + a + 16 * q, 16)]
                rows_flat[pl.ds(k * EMBED_DIM + 16 * q, 16)] = vals
        return carry

    lax.fori_loop(0, B_PER_W // 16, extract, None)
    pltpu.sync_copy(rows_flat, out_hbm.at[pl.ds(base * EMBED_DIM,
                                                B_PER_W * EMBED_DIM)])


def _copy2_body(x_ref, o1_ref, o2_ref):
    x = x_ref[...]
    o1_ref[...] = x
    o2_ref[...] = x


_copy2 = pl.pallas_call(
    _copy2_body,
    grid=(8,),
    in_specs=[pl.BlockSpec((EMBED_DIM, BATCH // 8), lambda i: (0, i))],
    out_specs=[pl.BlockSpec((EMBED_DIM, BATCH // 8), lambda i: (0, i))] * 2,
    out_shape=[jax.ShapeDtypeStruct((EMBED_DIM, BATCH), jnp.float32)] * 2,
)


def kernel(node_embedding, node_label, current_context, embeddings_weight):
    idx = node_label.astype(jnp.int32)
    table_t = embeddings_weight.T
    staged = _stage(table_t, table_t)
    out_flat = _sc_gather(staged, idx)
    o1T, o2T = _copy2(node_embedding.T)
    return (o1T.T, o2T.T, out_flat.reshape(BATCH, EMBED_DIM))


# speedup vs baseline: 1.5418x; 1.0529x over previous
import functools

import jax
import jax.numpy as jnp
from jax import lax
from jax.experimental import pallas as pl
from jax.experimental.pallas import tpu as pltpu
from jax.experimental.pallas import tpu_sc as plsc

EMBED_DIM = 64
BATCH = 16384
OP_ROWS = 1_000_000
NUM_CORES = 2
NUM_SUBCORES = 16
NUM_WORKERS = NUM_CORES * NUM_SUBCORES
B_PER_W = BATCH // NUM_WORKERS
CHUNK = 128
NUM_CHUNKS = B_PER_W // CHUNK

_TR_L = 16384
_GRID = 31
_S_HALF = _TR_L * _GRID

_MESH = plsc.VectorSubcoreMesh(
    core_axis_name="c", subcore_axis_name="s",
    num_cores=NUM_CORES, num_subcores=NUM_SUBCORES,
)


def _stage_body(xa_ref, xb_ref, o_ref):
    o_ref[...] = jnp.concatenate([xa_ref[...].T, xb_ref[...].T], axis=1)


_stage = pl.pallas_call(
    _stage_body,
    grid=(_GRID,),
    in_specs=[
        pl.BlockSpec((EMBED_DIM, _TR_L), lambda i: (0, i)),
        pl.BlockSpec((EMBED_DIM, _TR_L),
                     lambda i: (0, jnp.minimum(i + _GRID, 2 * _GRID - 2))),
    ],
    out_specs=pl.BlockSpec((_TR_L, 2 * EMBED_DIM), lambda i: (i, 0)),
    out_shape=jax.ShapeDtypeStruct((_S_HALF, 2 * EMBED_DIM), jnp.float32),
)


@functools.partial(
    pl.kernel,
    out_type=jax.ShapeDtypeStruct((BATCH * EMBED_DIM,), jnp.float32),
    mesh=_MESH,
    scratch_types=[
        pltpu.VMEM((B_PER_W,), jnp.int32),
        pltpu.VMEM((B_PER_W * 2 * EMBED_DIM,), jnp.float32),
        pltpu.VMEM((B_PER_W * EMBED_DIM,), jnp.float32),
        pltpu.SemaphoreType.DMA,
    ],
)
def _sc_gather(staged_hbm, idx_hbm, out_hbm,
               idx_v, pairs_flat, rows_flat, sem):
    wid = lax.axis_index("s") * NUM_CORES + lax.axis_index("c")
    base = wid * B_PER_W
    pltpu.sync_copy(idx_hbm.at[pl.ds(base, B_PER_W)], idx_v)

    def fire(g, carry):
        vec = idx_v[pl.ds(g * 16, 16)]
        rvec = jnp.where(vec >= _S_HALF, vec - _S_HALF, vec)
        for j in range(16):
            k = g * 16 + j
            pltpu.make_async_copy(
                staged_hbm.at[rvec[j]],
                pairs_flat.at[pl.ds(k * 2 * EMBED_DIM, 2 * EMBED_DIM)],
                sem,
            ).start()
        return carry

    def drain(g, carry):
        vec = idx_v[pl.ds(g * 16, 16)]
        rvec = jnp.where(vec >= _S_HALF, vec - _S_HALF, vec)
        for j in range(16):
            k = g * 16 + j
            pltpu.make_async_copy(
                staged_hbm.at[rvec[j]],
                pairs_flat.at[pl.ds(k * 2 * EMBED_DIM, 2 * EMBED_DIM)],
                sem,
            ).wait()
        return carry

    lax.fori_loop(0, B_PER_W // 16, fire, None)
    lax.fori_loop(0, B_PER_W // 16, drain, None)

    def extract(g, carry):
        vec = idx_v[pl.ds(g * 16, 16)]
        avec = jnp.where(vec >= _S_HALF, EMBED_DIM, 0)
        for j in range(16):
            k = g * 16 + j
            a = avec[j]
            for q in range(EMBED_DIM // 16):
                vals = pairs_flat[pl.ds(k * 2 * EMBED_DIM + a + 16 * q, 16)]
                rows_flat[pl.ds(k * EMBED_DIM + 16 * q, 16)] = vals
        return carry

    lax.fori_loop(0, B_PER_W // 16, extract, None)
    pltpu.sync_copy(rows_flat, out_hbm.at[pl.ds(base * EMBED_DIM,
                                                B_PER_W * EMBED_DIM)])


def _copy2_body(x_ref, o1_ref, o2_ref):
    x = x_ref[...]
    o1_ref[...] = x
    o2_ref[...] = x


_copy2 = pl.pallas_call(
    _copy2_body,
    grid=(8,),
    in_specs=[pl.BlockSpec((EMBED_DIM, BATCH // 8), lambda i: (0, i))],
    out_specs=[pl.BlockSpec((EMBED_DIM, BATCH // 8), lambda i: (0, i))] * 2,
    out_shape=[jax.ShapeDtypeStruct((EMBED_DIM, BATCH), jnp.float32)] * 2,
)


def kernel(node_embedding, node_label, current_context, embeddings_weight):
    idx = node_label.astype(jnp.int32)
    table_t = embeddings_weight.T
    staged = _stage(table_t, table_t)
    out_flat = _sc_gather(staged, idx)
    o1T, o2T = _copy2(node_embedding.T)
    return (o1T.T, o2T.T, out_flat.reshape(BATCH, EMBED_DIM))


# speedup vs baseline: 1.5434x; 1.0010x over previous
import functools

import jax
import jax.numpy as jnp
from jax import lax
from jax.experimental import pallas as pl
from jax.experimental.pallas import tpu as pltpu
from jax.experimental.pallas import tpu_sc as plsc

EMBED_DIM = 64
BATCH = 16384
OP_ROWS = 1_000_000
NUM_CORES = 2
NUM_SUBCORES = 16
NUM_WORKERS = NUM_CORES * NUM_SUBCORES
B_PER_W = BATCH // NUM_WORKERS
CHUNK = 128
NUM_CHUNKS = B_PER_W // CHUNK

_TR_L = 16384
_GRID = 31
_S_HALF = _TR_L * _GRID

_MESH = plsc.VectorSubcoreMesh(
    core_axis_name="c", subcore_axis_name="s",
    num_cores=NUM_CORES, num_subcores=NUM_SUBCORES,
)


def _stage_body(xa_ref, xb_ref, o_ref):
    o_ref[...] = jnp.concatenate([xa_ref[...].T, xb_ref[...].T], axis=1)


_stage = pl.pallas_call(
    _stage_body,
    grid=(_GRID,),
    in_specs=[
        pl.BlockSpec((EMBED_DIM, _TR_L), lambda i: (0, i)),
        pl.BlockSpec((EMBED_DIM, _TR_L),
                     lambda i: (0, jnp.minimum(i + _GRID,
                                               pl.cdiv(OP_ROWS, _TR_L) - 1))),
    ],
    out_specs=pl.BlockSpec((_TR_L, 2 * EMBED_DIM), lambda i: (i, 0)),
    out_shape=jax.ShapeDtypeStruct((_S_HALF, 2 * EMBED_DIM), jnp.float32),
)


@functools.partial(
    pl.kernel,
    out_type=jax.ShapeDtypeStruct((BATCH * EMBED_DIM,), jnp.float32),
    mesh=_MESH,
    scratch_types=[
        pltpu.VMEM((B_PER_W,), jnp.int32),
        pltpu.VMEM((B_PER_W * 2 * EMBED_DIM,), jnp.float32),
        pltpu.VMEM((B_PER_W * EMBED_DIM,), jnp.float32),
        pltpu.SemaphoreType.DMA,
    ],
)
def _sc_gather(staged_hbm, idx_hbm, out_hbm,
               idx_v, pairs_flat, rows_flat, sem):
    wid = lax.axis_index("s") * NUM_CORES + lax.axis_index("c")
    base = wid * B_PER_W
    pltpu.sync_copy(idx_hbm.at[pl.ds(base, B_PER_W)], idx_v)

    def fire(g, carry):
        vec = idx_v[pl.ds(g * 16, 16)]
        rvec = jnp.where(vec >= _S_HALF, vec - _S_HALF, vec)
        for j in range(16):
            k = g * 16 + j
            pltpu.make_async_copy(
                staged_hbm.at[rvec[j]],
                pairs_flat.at[pl.ds(k * 2 * EMBED_DIM, 2 * EMBED_DIM)],
                sem,
            ).start()
        return carry

    def drain(g, carry):
        vec = idx_v[pl.ds(g * 16, 16)]
        rvec = jnp.where(vec >= _S_HALF, vec - _S_HALF, vec)
        for j in range(16):
            k = g * 16 + j
            pltpu.make_async_copy(
                staged_hbm.at[rvec[j]],
                pairs_flat.at[pl.ds(k * 2 * EMBED_DIM, 2 * EMBED_DIM)],
                sem,
            ).wait()
        return carry

    lax.fori_loop(0, B_PER_W // 16, fire, None)
    lax.fori_loop(0, B_PER_W // 16, drain, None)

    def extract(g, carry):
        vec = idx_v[pl.ds(g * 16, 16)]
        avec = jnp.where(vec >= _S_HALF, EMBED_DIM, 0)
        for j in range(16):
            k = g * 16 + j
            a = avec[j]
            for q in range(EMBED_DIM // 16):
                vals = pairs_flat[pl.ds(k * 2 * EMBED_DIM + a + 16 * q, 16)]
                rows_flat[pl.ds(k * EMBED_DIM + 16 * q, 16)] = vals
        return carry

    lax.fori_loop(0, B_PER_W // 16, extract, None)
    pltpu.sync_copy(rows_flat, out_hbm.at[pl.ds(base * EMBED_DIM,
                                                B_PER_W * EMBED_DIM)])


def _copy2_body(x_ref, o1_ref, o2_ref):
    x = x_ref[...]
    o1_ref[...] = x
    o2_ref[...] = x


_copy2 = pl.pallas_call(
    _copy2_body,
    grid=(8,),
    in_specs=[pl.BlockSpec((EMBED_DIM, BATCH // 8), lambda i: (0, i))],
    out_specs=[pl.BlockSpec((EMBED_DIM, BATCH // 8), lambda i: (0, i))] * 2,
    out_shape=[jax.ShapeDtypeStruct((EMBED_DIM, BATCH), jnp.float32)] * 2,
)


def kernel(node_embedding, node_label, current_context, embeddings_weight):
    idx = node_label.astype(jnp.int32)
    table_t = embeddings_weight.T
    staged = _stage(table_t, table_t)
    out_flat = _sc_gather(staged, idx)
    o1T, o2T = _copy2(node_embedding.T)
    return (o1T.T, o2T.T, out_flat.reshape(BATCH, EMBED_DIM))
